# baseline (speedup 1.0000x reference)
import functools

import jax
import jax.numpy as jnp
from jax import lax
from jax.experimental import pallas as pl
from jax.experimental.pallas import tpu as pltpu
from jax.experimental.pallas import tpu_sc as plsc

N = 10000
E = 320000
D = 128
D_EMB = 16
H = 8

_EDGE_BLOCK = 6400
_NUM_CHUNKS = 10
_CHUNK = E // _NUM_CHUNKS
_NB_C = _CHUNK // _EDGE_BLOCK


def _linear1_body(x_ref, w_ref, o_ref):
    o_ref[...] = jnp.dot(x_ref[...], w_ref[...])


def _linear1(x, w):
    return pl.pallas_call(
        _linear1_body,
        out_shape=jax.ShapeDtypeStruct((N, D), jnp.float32),
    )(x, w)


_GATHER_WINDOW = 128


def _sc_gather(table, idx):
    n = idx.shape[0]
    idx2 = idx.reshape(1, n)
    mesh = plsc.VectorSubcoreMesh(core_axis_name="core",
                                  subcore_axis_name="subcore")

    @functools.partial(
        pl.kernel,
        out_type=jax.ShapeDtypeStruct((n, D), jnp.float32),
        mesh=mesh,
    )
    def gather_kernel(x_hbm, i_hbm, o_hbm):
        def body(i_vmem, o_vmem):
            pltpu.sync_copy(x_hbm.at[i_vmem.at[0]], o_vmem)

        pltpu.emit_pipeline(
            body,
            grid=(n // _GATHER_WINDOW,),
            in_specs=[pl.BlockSpec((1, _GATHER_WINDOW),
                                   index_map=lambda i: (0, i))],
            out_specs=[pl.BlockSpec((_GATHER_WINDOW, D),
                                    index_map=lambda i: (i, 0))],
            core_axis_name=("core", "subcore"),
            dimension_semantics=(pltpu.PARALLEL,),
        )(i_hbm, o_hbm)

    return gather_kernel(table, idx2)


def _edge_body(g_ref, emb_ref, a_ref, w0T_ref, w1T_ref, w2_ref, W2_ref,
               o_ref):
    h = jax.nn.silu(lax.dot_general(w0T_ref[...], emb_ref[...],
                                    (((1,), (1,)), ((), ()))))
    h = jax.nn.silu(jnp.dot(w1T_ref[...], h))
    t = lax.dot_general(h, w2_ref[...],
                        (((0,), (0,)), ((), ())))
    m = g_ref[...] * t * a_ref[...]
    o_ref[...] = jax.nn.silu(jnp.dot(m, W2_ref[...]))


def _edge_chunk_body(out_alias_ref, g_ref, emb_ref, a_ref, w0T_ref, w1T_ref,
                     w2_ref, W2_ref, o_ref):
    del out_alias_ref
    _edge_body(g_ref, emb_ref, a_ref, w0T_ref, w1T_ref, w2_ref, W2_ref,
               o_ref)


def _edge_kernel_chunk(outbuf, gathered_c, emb, attrs, w0T, w1T, w2, W2,
                       chunk):
    base = chunk * _NB_C
    common_in_specs = [
        pl.BlockSpec((_EDGE_BLOCK, D), lambda i: (i, 0)),
        pl.BlockSpec((_EDGE_BLOCK, D_EMB), lambda i, base=base: (base + i, 0)),
        pl.BlockSpec((_EDGE_BLOCK, 1), lambda i, base=base: (base + i, 0)),
        pl.BlockSpec((H, D_EMB), lambda i: (0, 0)),
        pl.BlockSpec((H, H), lambda i: (0, 0)),
        pl.BlockSpec((H, D), lambda i: (0, 0)),
        pl.BlockSpec((D, D), lambda i: (0, 0)),
    ]
    out_spec = pl.BlockSpec((_EDGE_BLOCK, D),
                            lambda i, base=base: (base + i, 0))
    out_shape = jax.ShapeDtypeStruct((E, D), jnp.float32)
    if outbuf is None:
        return pl.pallas_call(
            _edge_body,
            grid=(_NB_C,),
            in_specs=common_in_specs,
            out_specs=out_spec,
            out_shape=out_shape,
        )(gathered_c, emb, attrs, w0T, w1T, w2, W2)
    return pl.pallas_call(
        _edge_chunk_body,
        grid=(_NB_C,),
        in_specs=[pl.BlockSpec(memory_space=pl.ANY)] + common_in_specs,
        out_specs=out_spec,
        out_shape=out_shape,
        input_output_aliases={0: 0},
    )(outbuf, gathered_c, emb, attrs, w0T, w1T, w2, W2)


def kernel(node_features, edge_embedding, edge_attrs, edge_index,
           W1, mlp_w0, mlp_w1, mlp_w2, W2):
    edge_src = edge_index[0]
    node_feats = _linear1(node_features, W1)
    gathered = [
        _sc_gather(node_feats, lax.dynamic_slice_in_dim(edge_src, c * _CHUNK,
                                                        _CHUNK))
        for c in range(_NUM_CHUNKS)
    ]
    out = None
    for c in range(_NUM_CHUNKS):
        out = _edge_kernel_chunk(out, gathered[c], edge_embedding, edge_attrs,
                                 mlp_w0.T, mlp_w1.T, mlp_w2, W2, chunk=c)
    return out


# speedup vs baseline: 1.0713x; 1.0713x over previous
import functools

import jax
import jax.numpy as jnp
from jax import lax
from jax.experimental import pallas as pl
from jax.experimental.pallas import tpu as pltpu
from jax.experimental.pallas import tpu_sc as plsc

N = 10000
E = 320000
D = 128
D_EMB = 16
H = 8

_EDGE_BLOCK = 6400
_NUM_CHUNKS = 5
_CHUNK = E // _NUM_CHUNKS
_NB_C = _CHUNK // _EDGE_BLOCK


def _linear1_body(x_ref, w_ref, o_ref):
    o_ref[...] = jnp.dot(x_ref[...], w_ref[...])


def _linear1(x, w):
    return pl.pallas_call(
        _linear1_body,
        out_shape=jax.ShapeDtypeStruct((N, D), jnp.float32),
    )(x, w)


_GATHER_WINDOW = 128


def _sc_gather(table, idx):
    n = idx.shape[0]
    idx2 = idx.reshape(1, n)
    mesh = plsc.VectorSubcoreMesh(core_axis_name="core",
                                  subcore_axis_name="subcore")

    @functools.partial(
        pl.kernel,
        out_type=jax.ShapeDtypeStruct((n, D), jnp.float32),
        mesh=mesh,
    )
    def gather_kernel(x_hbm, i_hbm, o_hbm):
        def body(i_vmem, o_vmem):
            pltpu.sync_copy(x_hbm.at[i_vmem.at[0]], o_vmem)

        pltpu.emit_pipeline(
            body,
            grid=(n // _GATHER_WINDOW,),
            in_specs=[pl.BlockSpec((1, _GATHER_WINDOW),
                                   index_map=lambda i: (0, i))],
            out_specs=[pl.BlockSpec((_GATHER_WINDOW, D),
                                    index_map=lambda i: (i, 0))],
            core_axis_name=("core", "subcore"),
            dimension_semantics=(pltpu.PARALLEL,),
        )(i_hbm, o_hbm)

    return gather_kernel(table, idx2)


def _edge_body(g_ref, emb_ref, a_ref, w0T_ref, w1T_ref, w2_ref, W2_ref,
               o_ref):
    h = jax.nn.silu(lax.dot_general(w0T_ref[...], emb_ref[...],
                                    (((1,), (1,)), ((), ()))))
    h = jax.nn.silu(jnp.dot(w1T_ref[...], h))
    t = lax.dot_general(h, w2_ref[...],
                        (((0,), (0,)), ((), ())))
    m = g_ref[...] * t * a_ref[...]
    o_ref[...] = jax.nn.silu(jnp.dot(m, W2_ref[...]))


def _edge_chunk_body(out_alias_ref, g_ref, emb_ref, a_ref, w0T_ref, w1T_ref,
                     w2_ref, W2_ref, o_ref):
    del out_alias_ref
    _edge_body(g_ref, emb_ref, a_ref, w0T_ref, w1T_ref, w2_ref, W2_ref,
               o_ref)


def _edge_kernel_chunk(outbuf, gathered_c, emb, attrs, w0T, w1T, w2, W2,
                       chunk):
    base = chunk * _NB_C
    common_in_specs = [
        pl.BlockSpec((_EDGE_BLOCK, D), lambda i: (i, 0)),
        pl.BlockSpec((_EDGE_BLOCK, D_EMB), lambda i, base=base: (base + i, 0)),
        pl.BlockSpec((_EDGE_BLOCK, 1), lambda i, base=base: (base + i, 0)),
        pl.BlockSpec((H, D_EMB), lambda i: (0, 0)),
        pl.BlockSpec((H, H), lambda i: (0, 0)),
        pl.BlockSpec((H, D), lambda i: (0, 0)),
        pl.BlockSpec((D, D), lambda i: (0, 0)),
    ]
    out_spec = pl.BlockSpec((_EDGE_BLOCK, D),
                            lambda i, base=base: (base + i, 0))
    out_shape = jax.ShapeDtypeStruct((E, D), jnp.float32)
    if outbuf is None:
        return pl.pallas_call(
            _edge_body,
            grid=(_NB_C,),
            in_specs=common_in_specs,
            out_specs=out_spec,
            out_shape=out_shape,
        )(gathered_c, emb, attrs, w0T, w1T, w2, W2)
    return pl.pallas_call(
        _edge_chunk_body,
        grid=(_NB_C,),
        in_specs=[pl.BlockSpec(memory_space=pl.ANY)] + common_in_specs,
        out_specs=out_spec,
        out_shape=out_shape,
        input_output_aliases={0: 0},
    )(outbuf, gathered_c, emb, attrs, w0T, w1T, w2, W2)


def kernel(node_features, edge_embedding, edge_attrs, edge_index,
           W1, mlp_w0, mlp_w1, mlp_w2, W2):
    edge_src = edge_index[0]
    node_feats = _linear1(node_features, W1)
    gathered = [
        _sc_gather(node_feats, lax.dynamic_slice_in_dim(edge_src, c * _CHUNK,
                                                        _CHUNK))
        for c in range(_NUM_CHUNKS)
    ]
    out = None
    for c in range(_NUM_CHUNKS):
        out = _edge_kernel_chunk(out, gathered[c], edge_embedding, edge_attrs,
                                 mlp_w0.T, mlp_w1.T, mlp_w2, W2, chunk=c)
    return out


# speedup vs baseline: 1.0793x; 1.0075x over previous
import functools

import jax
import jax.numpy as jnp
from jax import lax
from jax.experimental import pallas as pl
from jax.experimental.pallas import tpu as pltpu
from jax.experimental.pallas import tpu_sc as plsc

N = 10000
E = 320000
D = 128
D_EMB = 16
H = 8

_EDGE_BLOCK = 8000
_NUM_CHUNKS = 4
_CHUNK = E // _NUM_CHUNKS
_NB_C = _CHUNK // _EDGE_BLOCK


def _linear1_body(x_ref, w_ref, o_ref):
    o_ref[...] = jnp.dot(x_ref[...], w_ref[...])


def _linear1(x, w):
    return pl.pallas_call(
        _linear1_body,
        out_shape=jax.ShapeDtypeStruct((N, D), jnp.float32),
    )(x, w)


_GATHER_WINDOW = 128


def _sc_gather(table, idx):
    n = idx.shape[0]
    idx2 = idx.reshape(1, n)
    mesh = plsc.VectorSubcoreMesh(core_axis_name="core",
                                  subcore_axis_name="subcore")

    @functools.partial(
        pl.kernel,
        out_type=jax.ShapeDtypeStruct((n, D), jnp.float32),
        mesh=mesh,
    )
    def gather_kernel(x_hbm, i_hbm, o_hbm):
        def body(i_vmem, o_vmem):
            pltpu.sync_copy(x_hbm.at[i_vmem.at[0]], o_vmem)

        pltpu.emit_pipeline(
            body,
            grid=(n // _GATHER_WINDOW,),
            in_specs=[pl.BlockSpec((1, _GATHER_WINDOW),
                                   index_map=lambda i: (0, i))],
            out_specs=[pl.BlockSpec((_GATHER_WINDOW, D),
                                    index_map=lambda i: (i, 0))],
            core_axis_name=("core", "subcore"),
            dimension_semantics=(pltpu.PARALLEL,),
        )(i_hbm, o_hbm)

    return gather_kernel(table, idx2)


def _edge_body(g_ref, emb_ref, a_ref, w0T_ref, w1T_ref, w2_ref, W2_ref,
               o_ref):
    h = jax.nn.silu(lax.dot_general(w0T_ref[...], emb_ref[...],
                                    (((1,), (1,)), ((), ()))))
    h = jax.nn.silu(jnp.dot(w1T_ref[...], h))
    t = lax.dot_general(h, w2_ref[...],
                        (((0,), (0,)), ((), ())))
    m = g_ref[...] * t * a_ref[...]
    o_ref[...] = jax.nn.silu(jnp.dot(m, W2_ref[...]))


def _edge_chunk_body(out_alias_ref, g_ref, emb_ref, a_ref, w0T_ref, w1T_ref,
                     w2_ref, W2_ref, o_ref):
    del out_alias_ref
    _edge_body(g_ref, emb_ref, a_ref, w0T_ref, w1T_ref, w2_ref, W2_ref,
               o_ref)


def _edge_kernel_chunk(outbuf, gathered_c, emb, attrs, w0T, w1T, w2, W2,
                       chunk):
    base = chunk * _NB_C
    common_in_specs = [
        pl.BlockSpec((_EDGE_BLOCK, D), lambda i: (i, 0)),
        pl.BlockSpec((_EDGE_BLOCK, D_EMB), lambda i, base=base: (base + i, 0)),
        pl.BlockSpec((_EDGE_BLOCK, 1), lambda i, base=base: (base + i, 0)),
        pl.BlockSpec((H, D_EMB), lambda i: (0, 0)),
        pl.BlockSpec((H, H), lambda i: (0, 0)),
        pl.BlockSpec((H, D), lambda i: (0, 0)),
        pl.BlockSpec((D, D), lambda i: (0, 0)),
    ]
    out_spec = pl.BlockSpec((_EDGE_BLOCK, D),
                            lambda i, base=base: (base + i, 0))
    out_shape = jax.ShapeDtypeStruct((E, D), jnp.float32)
    if outbuf is None:
        return pl.pallas_call(
            _edge_body,
            grid=(_NB_C,),
            in_specs=common_in_specs,
            out_specs=out_spec,
            out_shape=out_shape,
        )(gathered_c, emb, attrs, w0T, w1T, w2, W2)
    return pl.pallas_call(
        _edge_chunk_body,
        grid=(_NB_C,),
        in_specs=[pl.BlockSpec(memory_space=pl.ANY)] + common_in_specs,
        out_specs=out_spec,
        out_shape=out_shape,
        input_output_aliases={0: 0},
    )(outbuf, gathered_c, emb, attrs, w0T, w1T, w2, W2)


def kernel(node_features, edge_embedding, edge_attrs, edge_index,
           W1, mlp_w0, mlp_w1, mlp_w2, W2):
    edge_src = edge_index[0]
    node_feats = _linear1(node_features, W1)
    gathered = [
        _sc_gather(node_feats, lax.dynamic_slice_in_dim(edge_src, c * _CHUNK,
                                                        _CHUNK))
        for c in range(_NUM_CHUNKS)
    ]
    out = None
    for c in range(_NUM_CHUNKS):
        out = _edge_kernel_chunk(out, gathered[c], edge_embedding, edge_attrs,
                                 mlp_w0.T, mlp_w1.T, mlp_w2, W2, chunk=c)
    return out


# speedup vs baseline: 1.0992x; 1.0184x over previous
import functools

import jax
import jax.numpy as jnp
from jax import lax
from jax.experimental import pallas as pl
from jax.experimental.pallas import tpu as pltpu
from jax.experimental.pallas import tpu_sc as plsc

N = 10000
E = 320000
D = 128
D_EMB = 16
H = 8

_EDGE_BLOCK = 8000
_NUM_CHUNKS = 2
_CHUNK = E // _NUM_CHUNKS
_NB_C = _CHUNK // _EDGE_BLOCK


def _linear1_body(x_ref, w_ref, o_ref):
    o_ref[...] = jnp.dot(x_ref[...], w_ref[...])


def _linear1(x, w):
    return pl.pallas_call(
        _linear1_body,
        out_shape=jax.ShapeDtypeStruct((N, D), jnp.float32),
    )(x, w)


_GATHER_WINDOW = 128


def _sc_gather(table, idx):
    n = idx.shape[0]
    idx2 = idx.reshape(1, n)
    mesh = plsc.VectorSubcoreMesh(core_axis_name="core",
                                  subcore_axis_name="subcore")

    @functools.partial(
        pl.kernel,
        out_type=jax.ShapeDtypeStruct((n, D), jnp.float32),
        mesh=mesh,
    )
    def gather_kernel(x_hbm, i_hbm, o_hbm):
        def body(i_vmem, o_vmem):
            pltpu.sync_copy(x_hbm.at[i_vmem.at[0]], o_vmem)

        pltpu.emit_pipeline(
            body,
            grid=(n // _GATHER_WINDOW,),
            in_specs=[pl.BlockSpec((1, _GATHER_WINDOW),
                                   index_map=lambda i: (0, i))],
            out_specs=[pl.BlockSpec((_GATHER_WINDOW, D),
                                    index_map=lambda i: (i, 0))],
            core_axis_name=("core", "subcore"),
            dimension_semantics=(pltpu.PARALLEL,),
        )(i_hbm, o_hbm)

    return gather_kernel(table, idx2)


def _edge_body(g_ref, emb_ref, a_ref, w0T_ref, w1T_ref, w2_ref, W2_ref,
               o_ref):
    h = jax.nn.silu(lax.dot_general(w0T_ref[...], emb_ref[...],
                                    (((1,), (1,)), ((), ()))))
    h = jax.nn.silu(jnp.dot(w1T_ref[...], h))
    t = lax.dot_general(h, w2_ref[...],
                        (((0,), (0,)), ((), ())))
    m = g_ref[...] * t * a_ref[...]
    o_ref[...] = jax.nn.silu(jnp.dot(m, W2_ref[...]))


def _edge_chunk_body(out_alias_ref, g_ref, emb_ref, a_ref, w0T_ref, w1T_ref,
                     w2_ref, W2_ref, o_ref):
    del out_alias_ref
    _edge_body(g_ref, emb_ref, a_ref, w0T_ref, w1T_ref, w2_ref, W2_ref,
               o_ref)


def _edge_kernel_chunk(outbuf, gathered_c, emb, attrs, w0T, w1T, w2, W2,
                       chunk):
    base = chunk * _NB_C
    common_in_specs = [
        pl.BlockSpec((_EDGE_BLOCK, D), lambda i: (i, 0)),
        pl.BlockSpec((_EDGE_BLOCK, D_EMB), lambda i, base=base: (base + i, 0)),
        pl.BlockSpec((_EDGE_BLOCK, 1), lambda i, base=base: (base + i, 0)),
        pl.BlockSpec((H, D_EMB), lambda i: (0, 0)),
        pl.BlockSpec((H, H), lambda i: (0, 0)),
        pl.BlockSpec((H, D), lambda i: (0, 0)),
        pl.BlockSpec((D, D), lambda i: (0, 0)),
    ]
    out_spec = pl.BlockSpec((_EDGE_BLOCK, D),
                            lambda i, base=base: (base + i, 0))
    out_shape = jax.ShapeDtypeStruct((E, D), jnp.float32)
    if outbuf is None:
        return pl.pallas_call(
            _edge_body,
            grid=(_NB_C,),
            in_specs=common_in_specs,
            out_specs=out_spec,
            out_shape=out_shape,
        )(gathered_c, emb, attrs, w0T, w1T, w2, W2)
    return pl.pallas_call(
        _edge_chunk_body,
        grid=(_NB_C,),
        in_specs=[pl.BlockSpec(memory_space=pl.ANY)] + common_in_specs,
        out_specs=out_spec,
        out_shape=out_shape,
        input_output_aliases={0: 0},
    )(outbuf, gathered_c, emb, attrs, w0T, w1T, w2, W2)


def kernel(node_features, edge_embedding, edge_attrs, edge_index,
           W1, mlp_w0, mlp_w1, mlp_w2, W2):
    edge_src = edge_index[0]
    node_feats = _linear1(node_features, W1)
    gathered = [
        _sc_gather(node_feats, lax.dynamic_slice_in_dim(edge_src, c * _CHUNK,
                                                        _CHUNK))
        for c in range(_NUM_CHUNKS)
    ]
    out = None
    for c in range(_NUM_CHUNKS):
        out = _edge_kernel_chunk(out, gathered[c], edge_embedding, edge_attrs,
                                 mlp_w0.T, mlp_w1.T, mlp_w2, W2, chunk=c)
    return out


# speedup vs baseline: 1.1050x; 1.0052x over previous
import functools

import jax
import jax.numpy as jnp
from jax import lax
from jax.experimental import pallas as pl
from jax.experimental.pallas import tpu as pltpu
from jax.experimental.pallas import tpu_sc as plsc

N = 10000
E = 320000
D = 128
D_EMB = 16
H = 8

_EDGE_BLOCK = 8000
_NUM_CHUNKS = 2
_CHUNK = E // _NUM_CHUNKS
_NB_C = _CHUNK // _EDGE_BLOCK


def _linear1_body(x_ref, w_ref, o_ref):
    o_ref[...] = jnp.dot(x_ref[...], w_ref[...])


def _linear1(x, w):
    return pl.pallas_call(
        _linear1_body,
        out_shape=jax.ShapeDtypeStruct((N, D), jnp.float32),
    )(x, w)


_GATHER_WINDOW = 256


def _sc_gather(table, idx):
    n = idx.shape[0]
    idx2 = idx.reshape(1, n)
    mesh = plsc.VectorSubcoreMesh(core_axis_name="core",
                                  subcore_axis_name="subcore")

    @functools.partial(
        pl.kernel,
        out_type=jax.ShapeDtypeStruct((n, D), jnp.float32),
        mesh=mesh,
    )
    def gather_kernel(x_hbm, i_hbm, o_hbm):
        def body(i_vmem, o_vmem):
            pltpu.sync_copy(x_hbm.at[i_vmem.at[0]], o_vmem)

        pltpu.emit_pipeline(
            body,
            grid=(n // _GATHER_WINDOW,),
            in_specs=[pl.BlockSpec((1, _GATHER_WINDOW),
                                   index_map=lambda i: (0, i))],
            out_specs=[pl.BlockSpec((_GATHER_WINDOW, D),
                                    index_map=lambda i: (i, 0))],
            core_axis_name=("core", "subcore"),
            dimension_semantics=(pltpu.PARALLEL,),
        )(i_hbm, o_hbm)

    return gather_kernel(table, idx2)


def _edge_body(g_ref, emb_ref, a_ref, w0T_ref, w1T_ref, w2_ref, W2_ref,
               o_ref):
    h = jax.nn.silu(lax.dot_general(w0T_ref[...], emb_ref[...],
                                    (((1,), (1,)), ((), ()))))
    h = jax.nn.silu(jnp.dot(w1T_ref[...], h))
    t = lax.dot_general(h, w2_ref[...],
                        (((0,), (0,)), ((), ())))
    m = g_ref[...] * t * a_ref[...]
    o_ref[...] = jax.nn.silu(jnp.dot(m, W2_ref[...]))


def _edge_chunk_body(out_alias_ref, g_ref, emb_ref, a_ref, w0T_ref, w1T_ref,
                     w2_ref, W2_ref, o_ref):
    del out_alias_ref
    _edge_body(g_ref, emb_ref, a_ref, w0T_ref, w1T_ref, w2_ref, W2_ref,
               o_ref)


def _edge_kernel_chunk(outbuf, gathered_c, emb, attrs, w0T, w1T, w2, W2,
                       chunk):
    base = chunk * _NB_C
    common_in_specs = [
        pl.BlockSpec((_EDGE_BLOCK, D), lambda i: (i, 0)),
        pl.BlockSpec((_EDGE_BLOCK, D_EMB), lambda i, base=base: (base + i, 0)),
        pl.BlockSpec((_EDGE_BLOCK, 1), lambda i, base=base: (base + i, 0)),
        pl.BlockSpec((H, D_EMB), lambda i: (0, 0)),
        pl.BlockSpec((H, H), lambda i: (0, 0)),
        pl.BlockSpec((H, D), lambda i: (0, 0)),
        pl.BlockSpec((D, D), lambda i: (0, 0)),
    ]
    out_spec = pl.BlockSpec((_EDGE_BLOCK, D),
                            lambda i, base=base: (base + i, 0))
    out_shape = jax.ShapeDtypeStruct((E, D), jnp.float32)
    if outbuf is None:
        return pl.pallas_call(
            _edge_body,
            grid=(_NB_C,),
            in_specs=common_in_specs,
            out_specs=out_spec,
            out_shape=out_shape,
        )(gathered_c, emb, attrs, w0T, w1T, w2, W2)
    return pl.pallas_call(
        _edge_chunk_body,
        grid=(_NB_C,),
        in_specs=[pl.BlockSpec(memory_space=pl.ANY)] + common_in_specs,
        out_specs=out_spec,
        out_shape=out_shape,
        input_output_aliases={0: 0},
    )(outbuf, gathered_c, emb, attrs, w0T, w1T, w2, W2)


def kernel(node_features, edge_embedding, edge_attrs, edge_index,
           W1, mlp_w0, mlp_w1, mlp_w2, W2):
    edge_src = edge_index[0]
    node_feats = _linear1(node_features, W1)
    gathered = [
        _sc_gather(node_feats, lax.dynamic_slice_in_dim(edge_src, c * _CHUNK,
                                                        _CHUNK))
        for c in range(_NUM_CHUNKS)
    ]
    out = None
    for c in range(_NUM_CHUNKS):
        out = _edge_kernel_chunk(out, gathered[c], edge_embedding, edge_attrs,
                                 mlp_w0.T, mlp_w1.T, mlp_w2, W2, chunk=c)
    return out


# speedup vs baseline: 1.1218x; 1.0152x over previous
import functools

import jax
import jax.numpy as jnp
from jax import lax
from jax.experimental import pallas as pl
from jax.experimental.pallas import tpu as pltpu
from jax.experimental.pallas import tpu_sc as plsc

N = 10000
E = 320000
D = 128
D_EMB = 16
H = 8

_EDGE_BLOCK = 8000
_NUM_CHUNKS = 1
_CHUNK = E // _NUM_CHUNKS
_NB_C = _CHUNK // _EDGE_BLOCK


def _linear1_body(x_ref, w_ref, o_ref):
    o_ref[...] = jnp.dot(x_ref[...], w_ref[...])


def _linear1(x, w):
    return pl.pallas_call(
        _linear1_body,
        out_shape=jax.ShapeDtypeStruct((N, D), jnp.float32),
    )(x, w)


_GATHER_WINDOW = 256


def _sc_gather(table, idx):
    n = idx.shape[0]
    idx2 = idx.reshape(1, n)
    mesh = plsc.VectorSubcoreMesh(core_axis_name="core",
                                  subcore_axis_name="subcore")

    @functools.partial(
        pl.kernel,
        out_type=jax.ShapeDtypeStruct((n, D), jnp.float32),
        mesh=mesh,
    )
    def gather_kernel(x_hbm, i_hbm, o_hbm):
        def body(i_vmem, o_vmem):
            pltpu.sync_copy(x_hbm.at[i_vmem.at[0]], o_vmem)

        pltpu.emit_pipeline(
            body,
            grid=(n // _GATHER_WINDOW,),
            in_specs=[pl.BlockSpec((1, _GATHER_WINDOW),
                                   index_map=lambda i: (0, i))],
            out_specs=[pl.BlockSpec((_GATHER_WINDOW, D),
                                    index_map=lambda i: (i, 0))],
            core_axis_name=("core", "subcore"),
            dimension_semantics=(pltpu.PARALLEL,),
        )(i_hbm, o_hbm)

    return gather_kernel(table, idx2)


def _edge_body(g_ref, emb_ref, a_ref, w0T_ref, w1T_ref, w2_ref, W2_ref,
               o_ref):
    h = jax.nn.silu(lax.dot_general(w0T_ref[...], emb_ref[...],
                                    (((1,), (1,)), ((), ()))))
    h = jax.nn.silu(jnp.dot(w1T_ref[...], h))
    t = lax.dot_general(h, w2_ref[...],
                        (((0,), (0,)), ((), ())))
    m = g_ref[...] * t * a_ref[...]
    o_ref[...] = jax.nn.silu(jnp.dot(m, W2_ref[...]))


def _edge_chunk_body(out_alias_ref, g_ref, emb_ref, a_ref, w0T_ref, w1T_ref,
                     w2_ref, W2_ref, o_ref):
    del out_alias_ref
    _edge_body(g_ref, emb_ref, a_ref, w0T_ref, w1T_ref, w2_ref, W2_ref,
               o_ref)


def _edge_kernel_chunk(outbuf, gathered_c, emb, attrs, w0T, w1T, w2, W2,
                       chunk):
    base = chunk * _NB_C
    common_in_specs = [
        pl.BlockSpec((_EDGE_BLOCK, D), lambda i: (i, 0)),
        pl.BlockSpec((_EDGE_BLOCK, D_EMB), lambda i, base=base: (base + i, 0)),
        pl.BlockSpec((_EDGE_BLOCK, 1), lambda i, base=base: (base + i, 0)),
        pl.BlockSpec((H, D_EMB), lambda i: (0, 0)),
        pl.BlockSpec((H, H), lambda i: (0, 0)),
        pl.BlockSpec((H, D), lambda i: (0, 0)),
        pl.BlockSpec((D, D), lambda i: (0, 0)),
    ]
    out_spec = pl.BlockSpec((_EDGE_BLOCK, D),
                            lambda i, base=base: (base + i, 0))
    out_shape = jax.ShapeDtypeStruct((E, D), jnp.float32)
    if outbuf is None:
        return pl.pallas_call(
            _edge_body,
            grid=(_NB_C,),
            in_specs=common_in_specs,
            out_specs=out_spec,
            out_shape=out_shape,
        )(gathered_c, emb, attrs, w0T, w1T, w2, W2)
    return pl.pallas_call(
        _edge_chunk_body,
        grid=(_NB_C,),
        in_specs=[pl.BlockSpec(memory_space=pl.ANY)] + common_in_specs,
        out_specs=out_spec,
        out_shape=out_shape,
        input_output_aliases={0: 0},
    )(outbuf, gathered_c, emb, attrs, w0T, w1T, w2, W2)


def kernel(node_features, edge_embedding, edge_attrs, edge_index,
           W1, mlp_w0, mlp_w1, mlp_w2, W2):
    edge_src = edge_index[0]
    node_feats = _linear1(node_features, W1)
    gathered = [
        _sc_gather(node_feats, lax.dynamic_slice_in_dim(edge_src, c * _CHUNK,
                                                        _CHUNK))
        for c in range(_NUM_CHUNKS)
    ]
    out = None
    for c in range(_NUM_CHUNKS):
        out = _edge_kernel_chunk(out, gathered[c], edge_embedding, edge_attrs,
                                 mlp_w0.T, mlp_w1.T, mlp_w2, W2, chunk=c)
    return out


# speedup vs baseline: 2.0744x; 1.8492x over previous
import functools

import jax
import jax.numpy as jnp
from jax import lax
from jax.experimental import pallas as pl
from jax.experimental.pallas import tpu as pltpu
from jax.experimental.pallas import tpu_sc as plsc

N = 10000
E = 320000
D = 128
D_EMB = 16
H = 8

_EDGE_BLOCK = 16000
_NUM_CHUNKS = 1
_CHUNK = E // _NUM_CHUNKS
_NB_C = _CHUNK // _EDGE_BLOCK


def _linear1_body(x_ref, w_ref, o_ref):
    o_ref[...] = jnp.dot(x_ref[...], w_ref[...])


def _linear1(x, w):
    return pl.pallas_call(
        _linear1_body,
        out_shape=jax.ShapeDtypeStruct((N, D), jnp.float32),
    )(x, w)


_GATHER_WINDOW = 256


def _sc_gather(table, idx):
    n = idx.shape[0]
    idx2 = idx.reshape(1, n)
    mesh = plsc.VectorSubcoreMesh(core_axis_name="core",
                                  subcore_axis_name="subcore")

    @functools.partial(
        pl.kernel,
        out_type=jax.ShapeDtypeStruct((n, D), jnp.float32),
        mesh=mesh,
    )
    def gather_kernel(x_hbm, i_hbm, o_hbm):
        def body(i_vmem, o_vmem):
            pltpu.sync_copy(x_hbm.at[i_vmem.at[0]], o_vmem)

        pltpu.emit_pipeline(
            body,
            grid=(n // _GATHER_WINDOW,),
            in_specs=[pl.BlockSpec((1, _GATHER_WINDOW),
                                   index_map=lambda i: (0, i))],
            out_specs=[pl.BlockSpec((_GATHER_WINDOW, D),
                                    index_map=lambda i: (i, 0))],
            core_axis_name=("core", "subcore"),
            dimension_semantics=(pltpu.PARALLEL,),
        )(i_hbm, o_hbm)

    return gather_kernel(table, idx2)


def _edge_body(g_ref, emb_ref, a_ref, w0T_ref, w1T_ref, w2_ref, W2_ref,
               o_ref):
    h = jax.nn.silu(jnp.dot(w0T_ref[...], emb_ref[...]))
    h = jax.nn.silu(jnp.dot(w1T_ref[...], h))
    h = h * a_ref[...]
    t = lax.dot_general(h, w2_ref[...],
                        (((0,), (0,)), ((), ())))
    m = g_ref[...] * t
    o_ref[...] = jax.nn.silu(jnp.dot(m, W2_ref[...]))


def _edge_chunk_body(out_alias_ref, g_ref, emb_ref, a_ref, w0T_ref, w1T_ref,
                     w2_ref, W2_ref, o_ref):
    del out_alias_ref
    _edge_body(g_ref, emb_ref, a_ref, w0T_ref, w1T_ref, w2_ref, W2_ref,
               o_ref)


def _edge_kernel_chunk(outbuf, gathered_c, emb, attrs, w0T, w1T, w2, W2,
                       chunk):
    base = chunk * _NB_C
    common_in_specs = [
        pl.BlockSpec((_EDGE_BLOCK, D), lambda i: (i, 0)),
        pl.BlockSpec((D_EMB, _EDGE_BLOCK), lambda i, base=base: (0, base + i)),
        pl.BlockSpec((1, _EDGE_BLOCK), lambda i, base=base: (0, base + i)),
        pl.BlockSpec((H, D_EMB), lambda i: (0, 0)),
        pl.BlockSpec((H, H), lambda i: (0, 0)),
        pl.BlockSpec((H, D), lambda i: (0, 0)),
        pl.BlockSpec((D, D), lambda i: (0, 0)),
    ]
    out_spec = pl.BlockSpec((_EDGE_BLOCK, D),
                            lambda i, base=base: (base + i, 0))
    out_shape = jax.ShapeDtypeStruct((E, D), jnp.float32)
    if outbuf is None:
        return pl.pallas_call(
            _edge_body,
            grid=(_NB_C,),
            in_specs=common_in_specs,
            out_specs=out_spec,
            out_shape=out_shape,
        )(gathered_c, emb, attrs, w0T, w1T, w2, W2)
    return pl.pallas_call(
        _edge_chunk_body,
        grid=(_NB_C,),
        in_specs=[pl.BlockSpec(memory_space=pl.ANY)] + common_in_specs,
        out_specs=out_spec,
        out_shape=out_shape,
        input_output_aliases={0: 0},
    )(outbuf, gathered_c, emb, attrs, w0T, w1T, w2, W2)


def kernel(node_features, edge_embedding, edge_attrs, edge_index,
           W1, mlp_w0, mlp_w1, mlp_w2, W2):
    edge_src = edge_index[0]
    node_feats = _linear1(node_features, W1)
    gathered = [
        _sc_gather(node_feats, lax.dynamic_slice_in_dim(edge_src, c * _CHUNK,
                                                        _CHUNK))
        for c in range(_NUM_CHUNKS)
    ]
    emb_t = edge_embedding.T
    attrs_t = edge_attrs.reshape(1, E)
    out = None
    for c in range(_NUM_CHUNKS):
        out = _edge_kernel_chunk(out, gathered[c], emb_t, attrs_t,
                                 mlp_w0.T, mlp_w1.T, mlp_w2, W2, chunk=c)
    return out
